# baseline (speedup 1.0000x reference)
import functools
import math

import jax
import jax.numpy as jnp
from jax import lax
from jax.experimental import pallas as pl
from jax.experimental.pallas import tpu as pltpu
from jax.experimental.pallas import tpu_sc as plsc

N = 4096
NN = N * N
TOTAL = (N + 1) * N
DIST_MIN = 1.2 * 2.0 * math.sin(math.pi * 90.0 / 360.0)
DIST_MAX = 1.9 * 2.0
EPS1 = 0.1
EPS2 = 0.1

NC = 2
NS = 16
ZB = 32768


def _adj_sc_body(e_type_hbm, e_index_hbm, out_hbm,
                 r_v, c_v, t_v, idx2d, ones_v, zb_v, sem0, sem1):
    c = lax.axis_index("c")
    s = lax.axis_index("s")
    ept = e_type_hbm.shape[0] // NS

    def zfill(i, _):
        zb_v[pl.ds(i * 16, 16)] = jnp.zeros((16,), jnp.float32)
        return 0
    lax.fori_loop(0, ZB // 16, zfill, 0)
    for u in range(8):
        ones_v[pl.ds(u * 16, 16)] = jnp.ones((16,), jnp.float32)

    half = NN // NC
    slice_elems = half // NS
    base0 = c * half + s * slice_elems
    descs = []
    for q in range(slice_elems // ZB):
        descs.append(pltpu.async_copy(
            zb_v, out_hbm.at[pl.ds(base0 + q * ZB, ZB)], sem0))
    for dsc in descs:
        dsc.wait()
    plsc.subcore_barrier()

    ebase = s * ept
    pltpu.sync_copy(e_index_hbm.at[0, pl.ds(ebase, ept)], r_v)
    pltpu.sync_copy(e_index_hbm.at[1, pl.ds(ebase, ept)], c_v)
    pltpu.sync_copy(e_type_hbm.at[pl.ds(ebase, ept)], t_v)

    lo = c * half
    hi = lo + half
    trash = NN + c * 16

    nrow = (2 * ept) // 128
    hrow = nrow // 2

    def cbody(j, _):
        for u in range(8):
            off = j * 128 + u * 16
            r = r_v[pl.ds(off, 16)]
            cc = c_v[pl.ds(off, 16)]
            t = t_v[pl.ds(off, 16)]
            valid = t != 0
            fwd = r * N + cc
            rev = cc * N + r
            fidx = jnp.where(valid & (fwd >= lo) & (fwd < hi), fwd, trash)
            ridx = jnp.where(valid & (rev >= lo) & (rev < hi), rev, trash)
            idx2d[j, pl.ds(u * 16, 16)] = fidx
            idx2d[j + hrow, pl.ds(u * 16, 16)] = ridx
        return 0
    lax.fori_loop(0, hrow, cbody, 0)

    def sbody(g, _):
        ds_ = []
        for u in range(8):
            ds_.append(pltpu.async_copy(
                ones_v, out_hbm.at[idx2d.at[g * 8 + u]], sem1))
        for dsc in ds_:
            dsc.wait()
        return 0
    lax.fori_loop(0, nrow // 8, sbody, 0)


def _build_adj(e_type, e_index):
    mesh = plsc.VectorSubcoreMesh(core_axis_name="c", subcore_axis_name="s")
    ept = e_type.shape[0] // NS
    k = pl.kernel(
        _adj_sc_body,
        out_type=jax.ShapeDtypeStruct((TOTAL,), jnp.float32),
        mesh=mesh,
        scratch_types=[
            pltpu.VMEM((ept,), jnp.int32),
            pltpu.VMEM((ept,), jnp.int32),
            pltpu.VMEM((ept,), jnp.int32),
            pltpu.VMEM(((2 * ept) // 128, 128), jnp.int32),
            pltpu.VMEM((128,), jnp.float32),
            pltpu.VMEM((ZB,), jnp.float32),
            pltpu.SemaphoreType.DMA,
            pltpu.SemaphoreType.DMA,
        ],
    )
    return k(e_type, e_index)


def _bin_body(a_ref, o_ref):
    o_ref[...] = (a_ref[...] != 0.0).astype(jnp.bfloat16)


def _binarize(adjf):
    return pl.pallas_call(
        _bin_body,
        grid=(8,),
        in_specs=[pl.BlockSpec((512, N), lambda i: (i, 0))],
        out_specs=pl.BlockSpec((512, N), lambda i: (i, 0)),
        out_shape=jax.ShapeDtypeStruct((N, N), jnp.bfloat16),
    )(adjf)


BI = 1024
BJ = 1024
BK = 1024


def _mm_drift_body(a1_ref, a2_ref, xi_ref, xjt_ref, out_ref, acc_ref):
    i = pl.program_id(0)
    j = pl.program_id(1)
    k = pl.program_id(2)
    nk = pl.num_programs(2)

    @pl.when(k == 0)
    def _():
        acc_ref[...] = jnp.zeros_like(acc_ref)

    acc_ref[...] += jnp.dot(a1_ref[...], a2_ref[...],
                            preferred_element_type=jnp.float32)

    @pl.when((i == 0) & (j == 0) & (k == 0))
    def _():
        out_ref[0, 0] = 0.0

    @pl.when(k == nk - 1)
    def _():
        xi = xi_ref[...]
        xjt = xjt_ref[...]
        g = jnp.dot(xi, xjt, preferred_element_type=jnp.float32)
        ni = jnp.sum(xi * xi, axis=1, keepdims=True)
        nj = jnp.sum(xjt * xjt, axis=0, keepdims=True)
        d = jnp.sqrt(jnp.maximum(ni + nj - 2.0 * g, 0.0))
        mask = acc_ref[...] != 0.0
        ri = lax.broadcasted_iota(jnp.int32, (BI, BJ), 0) + i * BI
        ci = lax.broadcasted_iota(jnp.int32, (BI, BJ), 1) + j * BJ
        mask = mask & (ri != ci)
        drift = (EPS1 * jnp.maximum(d - DIST_MAX, 0.0)
                 + EPS2 * jnp.maximum(DIST_MIN - d, 0.0))
        out_ref[0, 0] += jnp.sum(jnp.where(mask, drift, 0.0))


def _mm_drift(adjb, xp, xpt):
    return pl.pallas_call(
        _mm_drift_body,
        grid=(N // BI, N // BJ, N // BK),
        in_specs=[
            pl.BlockSpec((BI, BK), lambda i, j, k: (i, k)),
            pl.BlockSpec((BK, BJ), lambda i, j, k: (k, j)),
            pl.BlockSpec((BI, 128), lambda i, j, k: (i, 0)),
            pl.BlockSpec((128, BJ), lambda i, j, k: (0, j)),
        ],
        out_specs=pl.BlockSpec((1, 1), lambda i, j, k: (0, 0),
                               memory_space=pltpu.SMEM),
        out_shape=jax.ShapeDtypeStruct((1, 1), jnp.float32),
        scratch_shapes=[pltpu.VMEM((BI, BJ), jnp.float32)],
    )(adjb, adjb, xp, xpt)


def kernel(x, e_type, e_index):
    adjf = _build_adj(e_type, e_index)
    adjb = _binarize(adjf.reshape(N + 1, N))
    xp = jnp.zeros((N, 128), x.dtype).at[:, :3].set(x)
    out = _mm_drift(adjb, xp, xp.T)
    return out[0, 0]
